# baseline (speedup 1.0000x reference)
import functools

import jax
import jax.numpy as jnp
from jax import lax
from jax.experimental import pallas as pl
from jax.experimental.pallas import tpu as pltpu
from jax.experimental.pallas import tpu_sc as plsc

N = 10000
E = 320000
D = 128
H = 128
C = 16
G = 64

NC = 2
NS = 16
NW = NC * NS
EPT = E // NW
CHUNK = 80
NCHUNK = EPT // CHUNK
NBLK = 4
NBLOCKS = (NCHUNK + NBLK - 1) // NBLK
ROWS_A = 632
ROWS_LAST = N - (NS - 1) * ROWS_A
NBUF = 4
PD = 2
ISLOTS = 4


def _mesh():
    return plsc.VectorSubcoreMesh(
        core_axis_name="c", subcore_axis_name="s",
        num_cores=NC, num_subcores=NS)


def _sc_params():
    return pltpu.CompilerParams(use_tc_tiling_on_sc=False)


def _striped(fn, s, base=0):
    @pl.when(s < NS - 1)
    def _():
        fn(base + s * ROWS_A, ROWS_A)

    @pl.when(s == NS - 1)
    def _():
        fn(base + (NS - 1) * ROWS_A, ROWS_LAST)


def _sc_hist_body(er_hbm, out0_hbm, out1_hbm, acc, didx, ones_v, zbuf,
                  h0, h1, h2, h3, h4):
    c = lax.axis_index("c")
    s = lax.axis_index("s")
    wid = s * NC + c
    hsem = [h0, h1, h2, h3, h4]

    @pl.loop(0, CHUNK)
    def _(i):
        ones_v[i, pl.ds(0, 16)] = jnp.ones((16,), jnp.float32)

    @pl.loop(0, N // NS // 5)
    def _(i):
        zbuf[i, pl.ds(0, 16)] = jnp.zeros((16,), jnp.float32)

    @pl.loop(0, 5)
    def _(k):
        pltpu.sync_copy(zbuf, acc.at[pl.ds(s * (N // NS) + k * 125, 125)])
    pltpu.sync_copy(er_hbm.at[1, wid], didx)
    plsc.subcore_barrier()

    @pl.loop(0, NCHUNK // 5)
    def _(o):
        for b in range(5):
            j = o * 5 + b
            pltpu.async_copy(ones_v, acc.at[didx.at[j]], hsem[b],
                             add=True)
        for b in range(5):
            pltpu.make_async_copy(er_hbm.at[1, 0, pl.ds(0, 16)], ones_v,
                                  hsem[b]).wait()

    plsc.subcore_barrier()

    def _copy_out(off, rows):
        @pl.when(c == 0)
        def _():
            pltpu.sync_copy(acc.at[pl.ds(off, rows)],
                            out0_hbm.at[pl.ds(off, rows)])

        @pl.when(c == 1)
        def _():
            pltpu.sync_copy(acc.at[pl.ds(off, rows)],
                            out1_hbm.at[pl.ds(off, rows)])

    _striped(_copy_out, s)


def _sc_hist(er):
    k = pl.kernel(
        _sc_hist_body,
        out_type=[jax.ShapeDtypeStruct((N, 16), jnp.float32),
                  jax.ShapeDtypeStruct((N, 16), jnp.float32)],
        mesh=_mesh(),
        scratch_types=[
            pltpu.VMEM_SHARED((N, 16), jnp.float32),
            pltpu.VMEM((NCHUNK, CHUNK), jnp.int32),
            pltpu.VMEM((CHUNK, 16), jnp.float32),
            pltpu.VMEM((125, 16), jnp.float32),
        ] + [pltpu.SemaphoreType.DMA] * 5,
        compiler_params=_sc_params(),
    )
    return k(er)


def _sc_agg_body(hp_hbm, er_hbm, out0_hbm, out1_hbm,
                 acc, ibuf, b0, b1, b2, b3,
                 g0, g1, g2, g3, s0, s1, s2, s3, isem):
    c = lax.axis_index("c")
    s = lax.axis_index("s")
    wid = s * NC + c
    bufs = [b0, b1, b2, b3]
    gsem = [g0, g1, g2, g3]
    ssem = [s0, s1, s2, s3]

    @pl.loop(0, CHUNK)
    def _(i):
        @pl.loop(0, D // 16)
        def _(k):
            b0[i, pl.ds(k * 16, 16)] = jnp.zeros((16,), jnp.float32)

    @pl.loop(0, 7)
    def _(k):
        pltpu.sync_copy(b0, acc.at[pl.ds(s * (N // NS) + k * CHUNK, CHUNK)])
    pltpu.sync_copy(b0.at[pl.ds(0, 65)],
                    acc.at[pl.ds(s * (N // NS) + 7 * CHUNK, 65)])

    for blk in range(ISLOTS):
        pltpu.sync_copy(er_hbm.at[0, wid, pl.ds(blk * NBLK, NBLK)],
                        ibuf.at[blk, 0])
        pltpu.sync_copy(er_hbm.at[1, wid, pl.ds(blk * NBLK, NBLK)],
                        ibuf.at[blk, 1])
    plsc.subcore_barrier()
    for b in range(PD):
        pltpu.async_copy(hp_hbm.at[ibuf.at[0, 0, b]], bufs[b], gsem[b])

    MAIN = NCHUNK - 1

    @pl.loop(0, MAIN // NBUF)
    def _(o):
        islot = lax.rem(o, ISLOTS)
        islot1 = lax.rem(o + 1, ISLOTS)
        for b in range(NBUF):
            j = o * NBUF + b
            jg = j + PD
            bg = (b + PD) % NBUF
            gslot = islot if b + PD < NBLK else islot1
            gpos = (b + PD) % NBLK

            @pl.when(jg < MAIN)
            def _():
                @pl.when(jg >= NBUF)
                def _():
                    pltpu.make_async_copy(hp_hbm.at[pl.ds(0, CHUNK)],
                                          bufs[bg], ssem[bg]).wait()
                pltpu.async_copy(hp_hbm.at[ibuf.at[gslot, 0, gpos]],
                                 bufs[bg], gsem[bg])

            if b == 1:
                @pl.when(jnp.logical_and(o >= 2, o <= NBLOCKS - 3))
                def _():
                    for _sd in range(2):
                        pltpu.make_async_copy(
                            er_hbm.at[0, 0, pl.ds(0, NBLK)],
                            ibuf.at[0, 0], isem).wait()

                @pl.when(jnp.logical_and(o >= 1, o <= NBLOCKS - 4))
                def _():
                    slot3 = lax.rem(o + 3, ISLOTS)
                    for sd in range(2):
                        pltpu.async_copy(
                            er_hbm.at[sd, wid, pl.ds((o + 3) * NBLK, NBLK)],
                            ibuf.at[slot3, sd], isem)

            pltpu.make_async_copy(hp_hbm.at[pl.ds(0, CHUNK)], bufs[b],
                                  gsem[b]).wait()
            pltpu.async_copy(bufs[b], acc.at[ibuf.at[islot, 1, b]],
                             ssem[b], add=True)

    pltpu.make_async_copy(hp_hbm.at[pl.ds(0, CHUNK)], bufs[0],
                          ssem[0]).wait()
    pltpu.async_copy(hp_hbm.at[ibuf.at[3, 0, 0]], bufs[0], gsem[0])
    pltpu.make_async_copy(hp_hbm.at[pl.ds(0, CHUNK)], bufs[0],
                          gsem[0]).wait()
    pltpu.async_copy(bufs[0], acc.at[ibuf.at[3, 1, 0]], ssem[0], add=True)
    for b in (1, 2, 3, 0):
        pltpu.make_async_copy(hp_hbm.at[pl.ds(0, CHUNK)], bufs[b],
                              ssem[b]).wait()

    plsc.subcore_barrier()

    def _copy_out(off, rows):
        @pl.when(c == 0)
        def _():
            pltpu.sync_copy(acc.at[pl.ds(off, rows)],
                            out0_hbm.at[pl.ds(off, rows)])

        @pl.when(c == 1)
        def _():
            pltpu.sync_copy(acc.at[pl.ds(off, rows)],
                            out1_hbm.at[pl.ds(off, rows)])

    _striped(_copy_out, s)


def _sc_agg(hp, er):
    k = pl.kernel(
        _sc_agg_body,
        out_type=[jax.ShapeDtypeStruct((N, D), jnp.float32),
                  jax.ShapeDtypeStruct((N, D), jnp.float32)],
        mesh=_mesh(),
        scratch_types=[
            pltpu.VMEM_SHARED((N, D), jnp.float32),
            pltpu.VMEM((ISLOTS, 2, NBLK, CHUNK), jnp.int32),
        ] + [pltpu.VMEM((CHUNK, D), jnp.float32)] * NBUF
          + [pltpu.SemaphoreType.DMA] * (2 * NBUF + 1),
        compiler_params=_sc_params(),
    )
    return k(hp, er)


def _tc_mm_body(x_ref, w_ref, o_ref):
    o_ref[...] = jnp.dot(x_ref[...], w_ref[...],
                         preferred_element_type=jnp.float32)


def _tc_mm(x, w):
    return pl.pallas_call(
        _tc_mm_body,
        out_shape=jax.ShapeDtypeStruct((x.shape[0], w.shape[1]), jnp.float32),
    )(x, w)


NB = 10
BN = N // NB


def _tc_scale_body(c0_ref, c1_ref, h_ref, hp_ref, dis_ref):
    deg = (c0_ref[...] + c1_ref[...] + 1.0)[:, 0:1]
    dis = lax.rsqrt(deg)
    dis_ref[...] = dis
    hp_ref[...] = dis * h_ref[...]


def _tc_scale(c0, c1, h):
    return pl.pallas_call(
        _tc_scale_body,
        grid=(NB,),
        in_specs=[
            pl.BlockSpec((BN, 16), lambda i: (i, 0)),
            pl.BlockSpec((BN, 16), lambda i: (i, 0)),
            pl.BlockSpec((BN, H), lambda i: (i, 0)),
        ],
        out_specs=[
            pl.BlockSpec((BN, H), lambda i: (i, 0)),
            pl.BlockSpec((BN, 1), lambda i: (i, 0)),
        ],
        out_shape=[
            jax.ShapeDtypeStruct((N, H), jnp.float32),
            jax.ShapeDtypeStruct((N, 1), jnp.float32),
        ],
    )(c0, c1, h)


def _tc_mid_body(p0_ref, p1_ref, hp_ref, dis_ref, b_ref, w_ref, hpn_ref,
                 z_ref, st_ref):
    ph = pl.program_id(0)
    i = pl.program_id(1)

    @pl.when(ph == 0)
    def _():
        @pl.when(i == 0)
        def _():
            st_ref[...] = jnp.zeros_like(st_ref)

        z = (dis_ref[...] * (p0_ref[...] + p1_ref[...] + hp_ref[...])
             + b_ref[...])
        z_ref[pl.ds(i * BN, BN), :] = z
        st_ref[0:1, :] += jnp.sum(z, axis=0, keepdims=True)
        st_ref[1:2, :] += jnp.sum(z * z, axis=0, keepdims=True)

    @pl.when(ph == 1)
    def _():
        z = z_ref[pl.ds(i * BN, BN), :]
        mu = st_ref[0:1, :] * (1.0 / N)
        var = st_ref[1:2, :] * (1.0 / N) - mu * mu
        zn = jnp.maximum((z - mu) * lax.rsqrt(var + 1e-5), 0.0)
        hn = jnp.dot(zn, w_ref[...], preferred_element_type=jnp.float32)
        hpn_ref[...] = dis_ref[...] * hn


def _tc_mid(p0, p1, hp, dis, b, w):
    node_block = pl.BlockSpec((BN, H), lambda ph, i: (i * (1 - ph), 0))
    return pl.pallas_call(
        _tc_mid_body,
        grid=(2, NB),
        in_specs=[
            node_block, node_block, node_block,
            pl.BlockSpec((BN, 1), lambda ph, i: (i, 0)),
            pl.BlockSpec((1, H), lambda ph, i: (0, 0)),
            pl.BlockSpec((H, H), lambda ph, i: (0, 0)),
        ],
        out_specs=pl.BlockSpec((BN, H), lambda ph, i: (i, 0)),
        out_shape=jax.ShapeDtypeStruct((N, H), jnp.float32),
        scratch_shapes=[
            pltpu.VMEM((N, H), jnp.float32),
            pltpu.VMEM((2, H), jnp.float32),
        ],
    )(p0, p1, hp, dis, b, w)


def _tc_final_body(p0_ref, p1_ref, hp_ref, dis_ref, b_ref, batch_ref,
                   wl_ref, bl_ref, o_ref):
    z = (dis_ref[...] * (p0_ref[...] + p1_ref[...] + hp_ref[...])
         + b_ref[...])
    seg = lax.broadcasted_iota(jnp.int32, (G, N), 0)
    onehot = (seg == batch_ref[...]).astype(jnp.float32)
    sums = jnp.dot(onehot, z, preferred_element_type=jnp.float32)
    cnt = jnp.sum(onehot, axis=1, keepdims=True)
    pooled = sums / jnp.maximum(cnt, 1.0)
    o_ref[...] = jnp.dot(pooled, wl_ref[...],
                         preferred_element_type=jnp.float32) + bl_ref[...]


def _tc_final(p0, p1, hp, dis, b, batch2d, wl, bl):
    return pl.pallas_call(
        _tc_final_body,
        out_shape=jax.ShapeDtypeStruct((G, C), jnp.float32),
    )(p0, p1, hp, dis, b, batch2d, wl, bl)


def kernel(x, edge_index, batch, W1, b1, W2, b2, W3, b3, Wl, bl):
    er = edge_index.reshape(2, NW, NCHUNK, CHUNK)

    cnt0, cnt1 = _sc_hist(er)
    h1 = _tc_mm(x, W1)
    hp1, dis = _tc_scale(cnt0, cnt1, h1)

    p10, p11 = _sc_agg(hp1, er)
    hp2 = _tc_mid(p10, p11, hp1, dis, b1.reshape(1, H), W2)

    p20, p21 = _sc_agg(hp2, er)
    hp3 = _tc_mid(p20, p21, hp2, dis, b2.reshape(1, H), W3)

    p30, p31 = _sc_agg(hp3, er)
    return _tc_final(p30, p31, hp3, dis, b3.reshape(1, H),
                     batch.reshape(1, N), Wl, bl.reshape(1, C))


# speedup vs baseline: 1.0422x; 1.0422x over previous
import functools

import jax
import jax.numpy as jnp
from jax import lax
from jax.experimental import pallas as pl
from jax.experimental.pallas import tpu as pltpu
from jax.experimental.pallas import tpu_sc as plsc

N = 10000
E = 320000
D = 128
H = 128
C = 16
G = 64

NC = 2
NS = 16
NW = NC * NS
EPT = E // NW
CHUNK = 80
NCHUNK = EPT // CHUNK
NBLK = 4
NBLOCKS = (NCHUNK + NBLK - 1) // NBLK
ROWS_A = 632
ROWS_LAST = N - (NS - 1) * ROWS_A
NBUF = 4
PD = 2
ISLOTS = 4


def _mesh():
    return plsc.VectorSubcoreMesh(
        core_axis_name="c", subcore_axis_name="s",
        num_cores=NC, num_subcores=NS)


def _sc_params():
    return pltpu.CompilerParams(use_tc_tiling_on_sc=False)


def _striped(fn, s, base=0):
    @pl.when(s < NS - 1)
    def _():
        fn(base + s * ROWS_A, ROWS_A)

    @pl.when(s == NS - 1)
    def _():
        fn(base + (NS - 1) * ROWS_A, ROWS_LAST)


def _sc_hist_body(er_hbm, out0_hbm, out1_hbm, acc, didx, ones_v, zbuf,
                  h0, h1, h2, h3, h4):
    c = lax.axis_index("c")
    s = lax.axis_index("s")
    wid = s * NC + c
    hsem = [h0, h1, h2, h3, h4]

    @pl.loop(0, CHUNK)
    def _(i):
        ones_v[i, pl.ds(0, 16)] = jnp.ones((16,), jnp.float32)

    @pl.loop(0, N // NS // 5)
    def _(i):
        zbuf[i, pl.ds(0, 16)] = jnp.zeros((16,), jnp.float32)

    @pl.loop(0, 5)
    def _(k):
        pltpu.sync_copy(zbuf, acc.at[pl.ds(s * (N // NS) + k * 125, 125)])
    pltpu.sync_copy(er_hbm.at[1, wid], didx)
    plsc.subcore_barrier()

    @pl.loop(0, NCHUNK // 5)
    def _(o):
        for b in range(5):
            j = o * 5 + b
            pltpu.async_copy(ones_v, acc.at[didx.at[j]], hsem[b],
                             add=True)
        for b in range(5):
            pltpu.make_async_copy(er_hbm.at[1, 0, pl.ds(0, 16)], ones_v,
                                  hsem[b]).wait()

    plsc.subcore_barrier()

    def _copy_out(off, rows):
        @pl.when(c == 0)
        def _():
            pltpu.sync_copy(acc.at[pl.ds(off, rows)],
                            out0_hbm.at[pl.ds(off, rows)])

        @pl.when(c == 1)
        def _():
            pltpu.sync_copy(acc.at[pl.ds(off, rows)],
                            out1_hbm.at[pl.ds(off, rows)])

    _striped(_copy_out, s)


def _sc_hist(er):
    k = pl.kernel(
        _sc_hist_body,
        out_type=[jax.ShapeDtypeStruct((N, 16), jnp.float32),
                  jax.ShapeDtypeStruct((N, 16), jnp.float32)],
        mesh=_mesh(),
        scratch_types=[
            pltpu.VMEM_SHARED((N, 16), jnp.float32),
            pltpu.VMEM((NCHUNK, CHUNK), jnp.int32),
            pltpu.VMEM((CHUNK, 16), jnp.float32),
            pltpu.VMEM((125, 16), jnp.float32),
        ] + [pltpu.SemaphoreType.DMA] * 5,
        compiler_params=_sc_params(),
    )
    return k(er)


def _sc_agg_body(hp_hbm, er_hbm, out0_hbm, out1_hbm,
                 acc, ibuf, b0, b1, b2, b3,
                 g0, g1, g2, g3, s0, s1, s2, s3, isem):
    c = lax.axis_index("c")
    s = lax.axis_index("s")
    wid = s * NC + c
    bufs = [b0, b1, b2, b3]
    gsem = [g0, g1, g2, g3]
    ssem = [s0, s1, s2, s3]

    @pl.loop(0, CHUNK)
    def _(i):
        @pl.loop(0, D // 16)
        def _(k):
            b0[i, pl.ds(k * 16, 16)] = jnp.zeros((16,), jnp.float32)

    @pl.loop(0, 7)
    def _(k):
        pltpu.sync_copy(b0, acc.at[pl.ds(s * (N // NS) + k * CHUNK, CHUNK)])
    pltpu.sync_copy(b0.at[pl.ds(0, 65)],
                    acc.at[pl.ds(s * (N // NS) + 7 * CHUNK, 65)])

    for blk in range(ISLOTS):
        pltpu.sync_copy(er_hbm.at[0, wid, pl.ds(blk * NBLK, NBLK)],
                        ibuf.at[blk, 0])
        pltpu.sync_copy(er_hbm.at[1, wid, pl.ds(blk * NBLK, NBLK)],
                        ibuf.at[blk, 1])
    plsc.subcore_barrier()
    for b in range(PD):
        pltpu.async_copy(hp_hbm.at[ibuf.at[0, 0, b]], bufs[b], gsem[b])

    MAIN = NCHUNK - 1

    @pl.loop(0, MAIN // NBUF)
    def _(o):
        islot = lax.rem(o, ISLOTS)
        islot1 = lax.rem(o + 1, ISLOTS)
        for b in range(NBUF):
            j = o * NBUF + b
            jg = j + PD
            bg = (b + PD) % NBUF
            gslot = islot if b + PD < NBLK else islot1
            gpos = (b + PD) % NBLK

            @pl.when(jg < MAIN)
            def _():
                @pl.when(jg >= NBUF)
                def _():
                    pltpu.make_async_copy(hp_hbm.at[pl.ds(0, CHUNK)],
                                          bufs[bg], ssem[bg]).wait()
                pltpu.async_copy(hp_hbm.at[ibuf.at[gslot, 0, gpos]],
                                 bufs[bg], gsem[bg])

            if b == 1:
                @pl.when(jnp.logical_and(o >= 2, o <= NBLOCKS - 3))
                def _():
                    for _sd in range(2):
                        pltpu.make_async_copy(
                            er_hbm.at[0, 0, pl.ds(0, NBLK)],
                            ibuf.at[0, 0], isem).wait()

                @pl.when(jnp.logical_and(o >= 1, o <= NBLOCKS - 4))
                def _():
                    slot3 = lax.rem(o + 3, ISLOTS)
                    for sd in range(2):
                        pltpu.async_copy(
                            er_hbm.at[sd, wid, pl.ds((o + 3) * NBLK, NBLK)],
                            ibuf.at[slot3, sd], isem)

            pltpu.make_async_copy(hp_hbm.at[pl.ds(0, CHUNK)], bufs[b],
                                  gsem[b]).wait()
            pltpu.async_copy(bufs[b], acc.at[ibuf.at[islot, 1, b]],
                             ssem[b], add=True)

    pltpu.make_async_copy(hp_hbm.at[pl.ds(0, CHUNK)], bufs[0],
                          ssem[0]).wait()
    pltpu.async_copy(hp_hbm.at[ibuf.at[3, 0, 0]], bufs[0], gsem[0])
    pltpu.make_async_copy(hp_hbm.at[pl.ds(0, CHUNK)], bufs[0],
                          gsem[0]).wait()
    pltpu.async_copy(bufs[0], acc.at[ibuf.at[3, 1, 0]], ssem[0], add=True)
    for b in (1, 2, 3, 0):
        pltpu.make_async_copy(hp_hbm.at[pl.ds(0, CHUNK)], bufs[b],
                              ssem[b]).wait()

    plsc.subcore_barrier()

    def _copy_out(off, rows):
        @pl.when(c == 0)
        def _():
            pltpu.sync_copy(acc.at[pl.ds(off, rows)],
                            out0_hbm.at[pl.ds(off, rows)])

        @pl.when(c == 1)
        def _():
            pltpu.sync_copy(acc.at[pl.ds(off, rows)],
                            out1_hbm.at[pl.ds(off, rows)])

    _striped(_copy_out, s)


def _sc_agg(hp, er):
    k = pl.kernel(
        _sc_agg_body,
        out_type=[jax.ShapeDtypeStruct((N, D), jnp.float32),
                  jax.ShapeDtypeStruct((N, D), jnp.float32)],
        mesh=_mesh(),
        scratch_types=[
            pltpu.VMEM_SHARED((N, D), jnp.float32),
            pltpu.VMEM((ISLOTS, 2, NBLK, CHUNK), jnp.int32),
        ] + [pltpu.VMEM((CHUNK, D), jnp.float32)] * NBUF
          + [pltpu.SemaphoreType.DMA] * (2 * NBUF + 1),
        compiler_params=_sc_params(),
    )
    return k(hp, er)


def _tc_mm_body(x_ref, w_ref, o_ref):
    o_ref[...] = jnp.dot(x_ref[...], w_ref[...],
                         preferred_element_type=jnp.float32)


def _tc_mm(x, w):
    return pl.pallas_call(
        _tc_mm_body,
        out_shape=jax.ShapeDtypeStruct((x.shape[0], w.shape[1]), jnp.float32),
    )(x, w)


NB = 10
BN = N // NB


def _tc_scale_body(c0_ref, c1_ref, h_ref, hp_ref, dis_ref):
    deg = (c0_ref[...] + c1_ref[...] + 1.0)[:, 0:1]
    dis = lax.rsqrt(deg)
    dis_ref[...] = dis
    hp_ref[...] = dis * h_ref[...]


def _tc_scale(c0, c1, h):
    return pl.pallas_call(
        _tc_scale_body,
        out_shape=[
            jax.ShapeDtypeStruct((N, H), jnp.float32),
            jax.ShapeDtypeStruct((N, 1), jnp.float32),
        ],
    )(c0, c1, h)


def _tc_mid_body(p0_ref, p1_ref, hp_ref, dis_ref, b_ref, w_ref, hpn_ref):
    dis = dis_ref[...]
    z = dis * (p0_ref[...] + p1_ref[...] + hp_ref[...]) + b_ref[...]
    mu = jnp.mean(z, axis=0, keepdims=True)
    zc = z - mu
    var = jnp.mean(zc * zc, axis=0, keepdims=True)
    zn = jnp.maximum(zc * lax.rsqrt(var + 1e-5), 0.0)
    hn = jnp.dot(zn, w_ref[...], preferred_element_type=jnp.float32)
    hpn_ref[...] = dis * hn


def _tc_mid(p0, p1, hp, dis, b, w):
    return pl.pallas_call(
        _tc_mid_body,
        out_shape=jax.ShapeDtypeStruct((N, H), jnp.float32),
    )(p0, p1, hp, dis, b, w)


def _tc_final_body(p0_ref, p1_ref, hp_ref, dis_ref, b_ref, batch_ref,
                   wl_ref, bl_ref, o_ref):
    z = (dis_ref[...] * (p0_ref[...] + p1_ref[...] + hp_ref[...])
         + b_ref[...])
    seg = lax.broadcasted_iota(jnp.int32, (G, N), 0)
    onehot = (seg == batch_ref[...]).astype(jnp.float32)
    sums = jnp.dot(onehot, z, preferred_element_type=jnp.float32)
    cnt = jnp.sum(onehot, axis=1, keepdims=True)
    pooled = sums / jnp.maximum(cnt, 1.0)
    o_ref[...] = jnp.dot(pooled, wl_ref[...],
                         preferred_element_type=jnp.float32) + bl_ref[...]


def _tc_final(p0, p1, hp, dis, b, batch2d, wl, bl):
    return pl.pallas_call(
        _tc_final_body,
        out_shape=jax.ShapeDtypeStruct((G, C), jnp.float32),
    )(p0, p1, hp, dis, b, batch2d, wl, bl)


def kernel(x, edge_index, batch, W1, b1, W2, b2, W3, b3, Wl, bl):
    er = edge_index.reshape(2, NW, NCHUNK, CHUNK)

    cnt0, cnt1 = _sc_hist(er)
    h1 = _tc_mm(x, W1)
    hp1, dis = _tc_scale(cnt0, cnt1, h1)

    p10, p11 = _sc_agg(hp1, er)
    hp2 = _tc_mid(p10, p11, hp1, dis, b1.reshape(1, H), W2)

    p20, p21 = _sc_agg(hp2, er)
    hp3 = _tc_mid(p20, p21, hp2, dis, b2.reshape(1, H), W3)

    p30, p31 = _sc_agg(hp3, er)
    return _tc_final(p30, p31, hp3, dis, b3.reshape(1, H),
                     batch.reshape(1, N), Wl, bl.reshape(1, C))


# speedup vs baseline: 1.0968x; 1.0524x over previous
import functools

import jax
import jax.numpy as jnp
from jax import lax
from jax.experimental import pallas as pl
from jax.experimental.pallas import tpu as pltpu
from jax.experimental.pallas import tpu_sc as plsc

N = 10000
E = 320000
D = 128
H = 128
C = 16
G = 64

NC = 2
NS = 16
NW = NC * NS
EPT = E // NW
CHUNK = 80
NCHUNK = EPT // CHUNK
NBLK = 4
NBLOCKS = (NCHUNK + NBLK - 1) // NBLK
ROWS_A = 632
ROWS_LAST = N - (NS - 1) * ROWS_A
NBUF = 4
PD = 2
ISLOTS = 4


def _mesh():
    return plsc.VectorSubcoreMesh(
        core_axis_name="c", subcore_axis_name="s",
        num_cores=NC, num_subcores=NS)


def _sc_params():
    return pltpu.CompilerParams(use_tc_tiling_on_sc=False)


def _striped(fn, s, base=0):
    @pl.when(s < NS - 1)
    def _():
        fn(base + s * ROWS_A, ROWS_A)

    @pl.when(s == NS - 1)
    def _():
        fn(base + (NS - 1) * ROWS_A, ROWS_LAST)


def _sc_hist_body(er_hbm, out0_hbm, out1_hbm, acc, didx, ones_v, zbuf,
                  h0, h1, h2, h3, h4):
    c = lax.axis_index("c")
    s = lax.axis_index("s")
    wid = s * NC + c
    hsem = [h0, h1, h2, h3, h4]

    @pl.loop(0, CHUNK)
    def _(i):
        ones_v[i, pl.ds(0, 16)] = jnp.ones((16,), jnp.float32)

    @pl.loop(0, N // NS // 5)
    def _(i):
        zbuf[i, pl.ds(0, 16)] = jnp.zeros((16,), jnp.float32)

    @pl.loop(0, 5)
    def _(k):
        pltpu.sync_copy(zbuf, acc.at[pl.ds(s * (N // NS) + k * 125, 125)])
    pltpu.sync_copy(er_hbm.at[1, wid], didx)
    plsc.subcore_barrier()

    @pl.loop(0, NCHUNK // 5)
    def _(o):
        for b in range(5):
            j = o * 5 + b
            pltpu.async_copy(ones_v, acc.at[didx.at[j]], hsem[b],
                             add=True)
        for b in range(5):
            pltpu.make_async_copy(er_hbm.at[1, 0, pl.ds(0, 16)], ones_v,
                                  hsem[b]).wait()

    plsc.subcore_barrier()

    def _copy_out(off, rows):
        @pl.when(c == 0)
        def _():
            pltpu.sync_copy(acc.at[pl.ds(off, rows)],
                            out0_hbm.at[pl.ds(off, rows)])

        @pl.when(c == 1)
        def _():
            pltpu.sync_copy(acc.at[pl.ds(off, rows)],
                            out1_hbm.at[pl.ds(off, rows)])

    _striped(_copy_out, s)


def _sc_hist(er):
    k = pl.kernel(
        _sc_hist_body,
        out_type=[jax.ShapeDtypeStruct((N, 16), jnp.float32),
                  jax.ShapeDtypeStruct((N, 16), jnp.float32)],
        mesh=_mesh(),
        scratch_types=[
            pltpu.VMEM_SHARED((N, 16), jnp.float32),
            pltpu.VMEM((NCHUNK, CHUNK), jnp.int32),
            pltpu.VMEM((CHUNK, 16), jnp.float32),
            pltpu.VMEM((125, 16), jnp.float32),
        ] + [pltpu.SemaphoreType.DMA] * 5,
        compiler_params=_sc_params(),
    )
    return k(er)


def _sc_agg_body(hp_hbm, er_hbm, out0_hbm, out1_hbm,
                 acc, ibuf, b0, b1, b2, b3,
                 g0, g1, g2, g3, s0, s1, s2, s3, isem):
    c = lax.axis_index("c")
    s = lax.axis_index("s")
    wid = s * NC + c
    bufs = [b0, b1, b2, b3]
    gsem = [g0, g1, g2, g3]
    ssem = [s0, s1, s2, s3]

    @pl.loop(0, CHUNK)
    def _(i):
        @pl.loop(0, D // 16)
        def _(k):
            b0[i, pl.ds(k * 16, 16)] = jnp.zeros((16,), jnp.float32)

    @pl.loop(0, 7)
    def _(k):
        pltpu.async_copy(b0, acc.at[pl.ds(s * (N // NS) + k * CHUNK, CHUNK)],
                         isem)
    pltpu.async_copy(b0.at[pl.ds(0, 65)],
                     acc.at[pl.ds(s * (N // NS) + 7 * CHUNK, 65)], isem)
    for blk in range(ISLOTS):
        pltpu.async_copy(er_hbm.at[0, wid, pl.ds(blk * NBLK, NBLK)],
                         ibuf.at[blk, 0], gsem[0])
        pltpu.async_copy(er_hbm.at[1, wid, pl.ds(blk * NBLK, NBLK)],
                         ibuf.at[blk, 1], gsem[0])
    for _k in range(2 * ISLOTS):
        pltpu.make_async_copy(er_hbm.at[0, 0, pl.ds(0, NBLK)],
                              ibuf.at[0, 0], gsem[0]).wait()

    @pl.loop(0, 7)
    def _(k):
        pltpu.make_async_copy(hp_hbm.at[pl.ds(0, CHUNK)],
                              acc.at[pl.ds(0, CHUNK)], isem).wait()
    pltpu.make_async_copy(hp_hbm.at[pl.ds(0, 65)], acc.at[pl.ds(0, 65)],
                          isem).wait()
    plsc.subcore_barrier()
    for b in range(PD):
        pltpu.async_copy(hp_hbm.at[ibuf.at[0, 0, b]], bufs[b], gsem[b])

    MAIN = NCHUNK - 1

    @pl.loop(0, MAIN // NBUF)
    def _(o):
        islot = lax.rem(o, ISLOTS)
        islot1 = lax.rem(o + 1, ISLOTS)
        for b in range(NBUF):
            j = o * NBUF + b
            jg = j + PD
            bg = (b + PD) % NBUF
            gslot = islot if b + PD < NBLK else islot1
            gpos = (b + PD) % NBLK

            @pl.when(jg < MAIN)
            def _():
                @pl.when(jg >= NBUF)
                def _():
                    pltpu.make_async_copy(hp_hbm.at[pl.ds(0, CHUNK)],
                                          bufs[bg], ssem[bg]).wait()
                pltpu.async_copy(hp_hbm.at[ibuf.at[gslot, 0, gpos]],
                                 bufs[bg], gsem[bg])

            if b == 1:
                @pl.when(jnp.logical_and(o >= 2, o <= NBLOCKS - 3))
                def _():
                    for _sd in range(2):
                        pltpu.make_async_copy(
                            er_hbm.at[0, 0, pl.ds(0, NBLK)],
                            ibuf.at[0, 0], isem).wait()

                @pl.when(jnp.logical_and(o >= 1, o <= NBLOCKS - 4))
                def _():
                    slot3 = lax.rem(o + 3, ISLOTS)
                    for sd in range(2):
                        pltpu.async_copy(
                            er_hbm.at[sd, wid, pl.ds((o + 3) * NBLK, NBLK)],
                            ibuf.at[slot3, sd], isem)

            pltpu.make_async_copy(hp_hbm.at[pl.ds(0, CHUNK)], bufs[b],
                                  gsem[b]).wait()
            pltpu.async_copy(bufs[b], acc.at[ibuf.at[islot, 1, b]],
                             ssem[b], add=True)

    pltpu.make_async_copy(hp_hbm.at[pl.ds(0, CHUNK)], bufs[0],
                          ssem[0]).wait()
    pltpu.async_copy(hp_hbm.at[ibuf.at[3, 0, 0]], bufs[0], gsem[0])
    pltpu.make_async_copy(hp_hbm.at[pl.ds(0, CHUNK)], bufs[0],
                          gsem[0]).wait()
    pltpu.async_copy(bufs[0], acc.at[ibuf.at[3, 1, 0]], ssem[0], add=True)
    for b in (1, 2, 3, 0):
        pltpu.make_async_copy(hp_hbm.at[pl.ds(0, CHUNK)], bufs[b],
                              ssem[b]).wait()

    plsc.subcore_barrier()

    def _copy_out(off, rows):
        @pl.when(c == 0)
        def _():
            pltpu.sync_copy(acc.at[pl.ds(off, rows)],
                            out0_hbm.at[pl.ds(off, rows)])

        @pl.when(c == 1)
        def _():
            pltpu.sync_copy(acc.at[pl.ds(off, rows)],
                            out1_hbm.at[pl.ds(off, rows)])

    _striped(_copy_out, s)


def _sc_agg(hp, er):
    k = pl.kernel(
        _sc_agg_body,
        out_type=[jax.ShapeDtypeStruct((N, D), jnp.float32),
                  jax.ShapeDtypeStruct((N, D), jnp.float32)],
        mesh=_mesh(),
        scratch_types=[
            pltpu.VMEM_SHARED((N, D), jnp.float32),
            pltpu.VMEM((ISLOTS, 2, NBLK, CHUNK), jnp.int32),
        ] + [pltpu.VMEM((CHUNK, D), jnp.float32)] * NBUF
          + [pltpu.SemaphoreType.DMA] * (2 * NBUF + 1),
        compiler_params=_sc_params(),
    )
    return k(hp, er)


def _tc_mm_body(x_ref, w_ref, o_ref):
    o_ref[...] = jnp.dot(x_ref[...], w_ref[...],
                         preferred_element_type=jnp.float32)


def _tc_mm(x, w):
    return pl.pallas_call(
        _tc_mm_body,
        out_shape=jax.ShapeDtypeStruct((x.shape[0], w.shape[1]), jnp.float32),
    )(x, w)


NB = 10
BN = N // NB


def _tc_scale_body(c0_ref, c1_ref, h_ref, hp_ref, dis_ref):
    deg = (c0_ref[...] + c1_ref[...] + 1.0)[:, 0:1]
    dis = lax.rsqrt(deg)
    dis_ref[...] = dis
    hp_ref[...] = dis * h_ref[...]


def _tc_scale(c0, c1, h):
    return pl.pallas_call(
        _tc_scale_body,
        out_shape=[
            jax.ShapeDtypeStruct((N, H), jnp.float32),
            jax.ShapeDtypeStruct((N, 1), jnp.float32),
        ],
    )(c0, c1, h)


def _tc_mid_body(p0_ref, p1_ref, hp_ref, dis_ref, b_ref, w_ref, hpn_ref):
    dis = dis_ref[...]
    z = dis * (p0_ref[...] + p1_ref[...] + hp_ref[...]) + b_ref[...]
    mu = jnp.mean(z, axis=0, keepdims=True)
    zc = z - mu
    var = jnp.mean(zc * zc, axis=0, keepdims=True)
    zn = jnp.maximum(zc * lax.rsqrt(var + 1e-5), 0.0)
    hn = jnp.dot(zn, w_ref[...], preferred_element_type=jnp.float32)
    hpn_ref[...] = dis * hn


def _tc_mid(p0, p1, hp, dis, b, w):
    return pl.pallas_call(
        _tc_mid_body,
        out_shape=jax.ShapeDtypeStruct((N, H), jnp.float32),
    )(p0, p1, hp, dis, b, w)


def _tc_final_body(p0_ref, p1_ref, hp_ref, dis_ref, b_ref, batch_ref,
                   wl_ref, bl_ref, o_ref):
    z = (dis_ref[...] * (p0_ref[...] + p1_ref[...] + hp_ref[...])
         + b_ref[...])
    seg = lax.broadcasted_iota(jnp.int32, (G, N), 0)
    onehot = (seg == batch_ref[...]).astype(jnp.float32)
    sums = jnp.dot(onehot, z, preferred_element_type=jnp.float32)
    cnt = jnp.sum(onehot, axis=1, keepdims=True)
    pooled = sums / jnp.maximum(cnt, 1.0)
    o_ref[...] = jnp.dot(pooled, wl_ref[...],
                         preferred_element_type=jnp.float32) + bl_ref[...]


def _tc_final(p0, p1, hp, dis, b, batch2d, wl, bl):
    return pl.pallas_call(
        _tc_final_body,
        out_shape=jax.ShapeDtypeStruct((G, C), jnp.float32),
    )(p0, p1, hp, dis, b, batch2d, wl, bl)


def kernel(x, edge_index, batch, W1, b1, W2, b2, W3, b3, Wl, bl):
    er = edge_index.reshape(2, NW, NCHUNK, CHUNK)

    cnt0, cnt1 = _sc_hist(er)
    h1 = _tc_mm(x, W1)
    hp1, dis = _tc_scale(cnt0, cnt1, h1)

    p10, p11 = _sc_agg(hp1, er)
    hp2 = _tc_mid(p10, p11, hp1, dis, b1.reshape(1, H), W2)

    p20, p21 = _sc_agg(hp2, er)
    hp3 = _tc_mid(p20, p21, hp2, dis, b2.reshape(1, H), W3)

    p30, p31 = _sc_agg(hp3, er)
    return _tc_final(p30, p31, hp3, dis, b3.reshape(1, H),
                     batch.reshape(1, N), Wl, bl.reshape(1, C))
